# baseline (speedup 1.0000x reference)
import functools

import jax
import jax.numpy as jnp
from jax import lax
from jax.experimental import pallas as pl
from jax.experimental.pallas import tpu as pltpu
from jax.experimental.pallas import tpu_sc as plsc

N_VALUES = 1000
LANES = 16
CHUNK = 128


@functools.cache
def _make_planes(V, D):
    W = 8 * 128
    ng = -(-V // W)

    def body(in_ref, *out_refs):
        block = in_ref[...]
        for j in range(D):
            out_refs[j][...] = block[j, :]

    return pl.pallas_call(
        body,
        grid=(ng,),
        in_specs=[pl.BlockSpec((D, W), lambda g: (0, g))],
        out_specs=[pl.BlockSpec((W,), lambda g: (g,)) for _ in range(D)],
        out_shape=[jax.ShapeDtypeStruct((ng * W,), jnp.int32)
                   for _ in range(D)],
    )


@functools.cache
def _make_gather(B, D):
    info = plsc.get_sparse_core_info()
    nw = info.num_cores * info.num_subcores
    b_per_w = B // nw
    n_chunks = b_per_w // CHUNK
    mesh = plsc.VectorSubcoreMesh(core_axis_name="c", subcore_axis_name="s")

    @functools.partial(
        pl.kernel,
        mesh=mesh,
        out_type=jax.ShapeDtypeStruct((B * 8,), jnp.int32),
        compiler_params=pltpu.CompilerParams(
            use_tc_tiling_on_sc=False, needs_layout_passes=False),
        scratch_types=[
            pltpu.VMEM((b_per_w,), jnp.int32),
            pltpu.VMEM((b_per_w,), jnp.int32),
            pltpu.VMEM((n_chunks, CHUNK), jnp.int32),
            pltpu.VMEM((D, b_per_w), jnp.int32),
            pltpu.VMEM((b_per_w * 8,), jnp.int32),
            pltpu.SemaphoreType.DMA,
        ],
    )
    def gather_kernel(x_hbm, sh_hbm, *rest):
        plane_hbms = rest[:D]
        out_hbm = rest[D]
        x_v, sh_v, idx_v, stage_v, rows_v, sem = rest[D + 1:]
        wid = lax.axis_index("s") * info.num_cores + lax.axis_index("c")
        base = wid * b_per_w
        pltpu.sync_copy(x_hbm.at[pl.ds(base, b_per_w)], x_v)
        pltpu.sync_copy(sh_hbm.at[pl.ds(base, b_per_w)], sh_v)
        for c in range(n_chunks):
            for v in range(CHUNK // LANES):
                sl = pl.ds(c * CHUNK + v * LANES, LANES)
                idx_v[c, pl.ds(v * LANES, LANES)] = (
                    x_v[sl] * N_VALUES + sh_v[sl])
        copies = []
        for j in range(D):
            for c in range(n_chunks):
                cp = pltpu.make_async_copy(
                    plane_hbms[j].at[idx_v.at[c]],
                    stage_v.at[j, pl.ds(c * CHUNK, CHUNK)], sem)
                cp.start()
                copies.append(cp)
        for cp in copies:
            cp.wait()
        lane8 = lax.iota(jnp.int32, LANES) * 8
        for j in range(D):
            for v in range(b_per_w // LANES):
                vals = stage_v[j, pl.ds(v * LANES, LANES)] + 1
                plsc.store_scatter(rows_v, [lane8 + (v * LANES * 8 + j)], vals)
        pltpu.sync_copy(rows_v, out_hbm.at[pl.ds(base * 8, b_per_w * 8)])

    return gather_kernel


def kernel(x, mapping):
    batch = x.shape[0]
    V, D = mapping.shape
    key = jax.random.key(42)
    shifts = jnp.stack(
        [jax.random.randint(jax.random.fold_in(key, i), (batch,), 0, N_VALUES,
                            dtype=x.dtype) for i in range(2)],
        axis=1)
    planes = _make_planes(V, D)(mapping.T)
    out8 = _make_gather(2 * batch, D)(
        x.reshape(-1), shifts.reshape(-1), *planes).reshape(2 * batch, 8)
    result = out8[:, :D].reshape(batch, 2 * D)
    zeros = jnp.zeros(result.shape, jnp.float32)
    return (result, zeros, zeros)


# speedup vs baseline: 4.3819x; 4.3819x over previous
import functools

import jax
import jax.numpy as jnp
from jax import lax
from jax.experimental import pallas as pl
from jax.experimental.pallas import tpu as pltpu
from jax.experimental.pallas import tpu_sc as plsc

N_VALUES = 1000
LANES = 16
CHUNK = 128


@functools.cache
def _make_planes(V, D):
    W = 512 * 128
    ng = -(-V // W)

    def body(in_ref, *out_refs):
        block = in_ref[...]
        for j in range(D):
            out_refs[j][...] = block[j, :]

    return pl.pallas_call(
        body,
        grid=(ng,),
        in_specs=[pl.BlockSpec((D, W), lambda g: (0, g))],
        out_specs=[pl.BlockSpec((W,), lambda g: (g,)) for _ in range(D)],
        out_shape=[jax.ShapeDtypeStruct((ng * W,), jnp.int32)
                   for _ in range(D)],
    )


@functools.cache
def _make_gather(B, D):
    info = plsc.get_sparse_core_info()
    nw = info.num_cores * info.num_subcores
    b_per_w = B // nw
    n_chunks = b_per_w // CHUNK
    mesh = plsc.VectorSubcoreMesh(core_axis_name="c", subcore_axis_name="s")

    @functools.partial(
        pl.kernel,
        mesh=mesh,
        out_type=jax.ShapeDtypeStruct((B * 8,), jnp.int32),
        compiler_params=pltpu.CompilerParams(
            use_tc_tiling_on_sc=False, needs_layout_passes=False),
        scratch_types=[
            pltpu.VMEM((b_per_w,), jnp.int32),
            pltpu.VMEM((b_per_w,), jnp.int32),
            pltpu.VMEM((n_chunks, CHUNK), jnp.int32),
            pltpu.VMEM((D, b_per_w), jnp.int32),
            pltpu.VMEM((b_per_w * 8,), jnp.int32),
            pltpu.SemaphoreType.DMA,
        ],
    )
    def gather_kernel(x_hbm, sh_hbm, *rest):
        plane_hbms = rest[:D]
        out_hbm = rest[D]
        x_v, sh_v, idx_v, stage_v, rows_v, sem = rest[D + 1:]
        wid = lax.axis_index("s") * info.num_cores + lax.axis_index("c")
        base = wid * b_per_w
        pltpu.sync_copy(x_hbm.at[pl.ds(base, b_per_w)], x_v)
        pltpu.sync_copy(sh_hbm.at[pl.ds(base, b_per_w)], sh_v)
        for c in range(n_chunks):
            for v in range(CHUNK // LANES):
                sl = pl.ds(c * CHUNK + v * LANES, LANES)
                idx_v[c, pl.ds(v * LANES, LANES)] = (
                    x_v[sl] * N_VALUES + sh_v[sl])
        copies = []
        for j in range(D):
            for c in range(n_chunks):
                cp = pltpu.make_async_copy(
                    plane_hbms[j].at[idx_v.at[c]],
                    stage_v.at[j, pl.ds(c * CHUNK, CHUNK)], sem)
                cp.start()
                copies.append(cp)
        for cp in copies:
            cp.wait()
        lane8 = lax.iota(jnp.int32, LANES) * 8
        for j in range(D):
            for v in range(b_per_w // LANES):
                vals = stage_v[j, pl.ds(v * LANES, LANES)] + 1
                plsc.store_scatter(rows_v, [lane8 + (v * LANES * 8 + j)], vals)
        pltpu.sync_copy(rows_v, out_hbm.at[pl.ds(base * 8, b_per_w * 8)])

    return gather_kernel


def kernel(x, mapping):
    batch = x.shape[0]
    V, D = mapping.shape
    key = jax.random.key(42)
    shifts = jnp.stack(
        [jax.random.randint(jax.random.fold_in(key, i), (batch,), 0, N_VALUES,
                            dtype=x.dtype) for i in range(2)],
        axis=1)
    planes = _make_planes(V, D)(mapping.T)
    out8 = _make_gather(2 * batch, D)(
        x.reshape(-1), shifts.reshape(-1), *planes).reshape(2 * batch, 8)
    result = out8[:, :D].reshape(batch, 2 * D)
    zeros = jnp.zeros(result.shape, jnp.float32)
    return (result, zeros, zeros)
